# baseline (speedup 1.0000x reference)
import jax
import jax.numpy as jnp
from jax import lax
from jax.experimental import pallas as pl
from jax.experimental.pallas import tpu as pltpu
from jax.experimental.pallas import tpu_sc as plsc

BATCH = 4096
SEQ = 200
EMBED = 64
VOCAB2 = 1000002

NC = 2
NS = 16
NW = NC * NS
BBLK = BATCH // NW
NB = 2

TCBLK = 512
NTCB = (VOCAB2 + TCBLK - 1) // TCBLK
DUP_ROWS = NTCB * TCBLK

_mesh = plsc.VectorSubcoreMesh(core_axis_name="c", subcore_axis_name="s")


def _dup_body(t_ref, o_ref):
    xt = t_ref[...].T
    o_ref[...] = jnp.concatenate([xt, xt], axis=1)


def _dupify(tT):
    return pl.pallas_call(
        _dup_body,
        grid=(NTCB,),
        in_specs=[pl.BlockSpec((EMBED, TCBLK), lambda i: (0, i))],
        out_specs=pl.BlockSpec((TCBLK, 128), lambda i: (i, 0)),
        out_shape=jax.ShapeDtypeStruct((DUP_ROWS, 128), jnp.float32),
    )(tT)


def _embed_body(wT, tab2, out, idxv, bA, bB, tA, tB, gA, gB, wvA, wvB):
    bufs = (bA, bB)
    bufTs = (tA, tB)
    gsems = (gA, gB)
    wsems = (wvA, wvB)
    wid = lax.axis_index("s") * NC + lax.axis_index("c")
    b0 = wid * BBLK

    pltpu.sync_copy(wT.at[:, pl.ds(b0, BBLK)], idxv)

    iota16 = lax.iota(jnp.int32, 16)
    skews = [lax.bitwise_and(iota16 + c, 15) for c in range(16)]

    def transpose_group(j):
        @plsc.parallel_loop(0, (EMBED // 16) * (BBLK // 16), unroll=4)
        def subtile(t):
            d0 = lax.mul(lax.bitwise_and(t, EMBED // 16 - 1), 16)
            ld_rows = iota16 + lax.mul(
                lax.shift_right_logical(t, 2), 16
            )
            for c in range(16):
                dcols = skews[c] + d0
                val = plsc.load_gather(bufs[j], [ld_rows, dcols])
                plsc.store_scatter(bufTs[j], [dcols, ld_rows], val)

    for j in range(NB):
        pltpu.async_copy(tab2.at[idxv.at[j]], bufs[j], gsems[j])

    def step(t, carry):
        for j in range(NB):
            s = t * NB + j
            pltpu.make_async_copy(tab2.at[idxv.at[s]], bufs[j], gsems[j]).wait()

            @pl.when(s >= NB)
            def _():
                pltpu.make_async_copy(
                    bufTs[j], out.at[s - NB, :, pl.ds(b0, BBLK)], wsems[j]
                ).wait()

            transpose_group(j)
            pltpu.async_copy(bufTs[j], out.at[s, :, pl.ds(b0, BBLK)], wsems[j])

            @pl.when(s + NB < SEQ)
            def _():
                pltpu.async_copy(tab2.at[idxv.at[s + NB]], bufs[j], gsems[j])
        return carry

    lax.fori_loop(0, SEQ // NB, step, 0)

    for j in range(NB):
        s_last = SEQ - NB + j
        pltpu.make_async_copy(
            bufTs[j], out.at[s_last, :, pl.ds(b0, BBLK)], wsems[j]
        ).wait()


@jax.jit
def _embed(wT, tab2):
    run = pl.kernel(
        _embed_body,
        out_type=jax.ShapeDtypeStruct((SEQ, EMBED, BATCH), jnp.float32),
        mesh=_mesh,
        scratch_types=[
            pltpu.VMEM((SEQ, BBLK), jnp.int32),
        ]
        + [pltpu.VMEM((BBLK, 128), jnp.float32) for _ in range(NB)]
        + [pltpu.VMEM((EMBED, BBLK), jnp.float32) for _ in range(NB)]
        + [pltpu.SemaphoreType.DMA] * (2 * NB),
        compiler_params=pltpu.CompilerParams(needs_layout_passes=False),
    )
    return run(wT, tab2)


def kernel(word_batch, table):
    wT = word_batch.astype(jnp.int32).T
    tableT = table.T
    tab2 = _dupify(tableT)
    outT = _embed(wT, tab2)
    return jnp.transpose(outT, (2, 0, 1))


# speedup vs baseline: 2.2112x; 2.2112x over previous
import jax
import jax.numpy as jnp
from jax import lax
from jax.experimental import pallas as pl
from jax.experimental.pallas import tpu as pltpu
from jax.experimental.pallas import tpu_sc as plsc

BATCH = 4096
SEQ = 200
EMBED = 64
VOCAB2 = 1000002

NC = 2
NS = 16
NW = NC * NS
BBLK = BATCH // NW
NB = 2

NBLK_FULL = (VOCAB2 // 2) // 64
TAIL_V = NBLK_FULL * 128
PAIR_PAD = NBLK_FULL * 64 + 40

_mesh = plsc.VectorSubcoreMesh(core_axis_name="c", subcore_axis_name="s")


def _iota_skews():
    iota16 = lax.iota(jnp.int32, 16)
    return iota16, [lax.bitwise_and(iota16 + c, 15) for c in range(16)]


def _k1_body(tT, tailp, tab2, sA, sB, pA, pB, tailv, rA, rB, wvA, wvB):
    slabs = (sA, sB)
    slabPs = (pA, pB)
    rsems = (rA, rB)
    wsems = (wvA, wvB)
    wid = lax.axis_index("s") * NC + lax.axis_index("c")
    iota16, skews = _iota_skews()
    pcols = [
        lax.shift_left(lax.bitwise_and(skews[c], 1), 6) + iota16
        for c in range(16)
    ]

    def transpose_blk(j):
        @plsc.parallel_loop(0, 32, unroll=4)
        def subtile(t):
            r0 = lax.mul(lax.bitwise_and(t, 3), 16)
            c0 = lax.mul(lax.shift_right_logical(t, 2), 16)
            ld_rows = iota16 + r0
            for c in range(16):
                vcols = skews[c] + c0
                val = plsc.load_gather(slabs[j], [ld_rows, vcols])
                st_rows = lax.shift_right_logical(vcols, 1)
                st_cols = pcols[c] + r0
                plsc.store_scatter(slabPs[j], [st_rows, st_cols], val)

    for j in range(2):
        pltpu.async_copy(
            tT.at[:, pl.ds((32 * j + wid) * 128, 128)], slabs[j], rsems[j]
        )

    def step(tt, c):
        for j in range(2):
            t = 2 * tt + j
            blk = 32 * t + wid

            @pl.when(blk < NBLK_FULL)
            def _():
                pltpu.make_async_copy(
                    tT.at[:, pl.ds(blk * 128, 128)], slabs[j], rsems[j]
                ).wait()

                @pl.when(t >= 2)
                def _():
                    pltpu.make_async_copy(
                        slabPs[j], tab2.at[pl.ds((blk - 64) * 64, 64)], wsems[j]
                    ).wait()

                transpose_blk(j)
                pltpu.async_copy(
                    slabPs[j], tab2.at[pl.ds(blk * 64, 64)], wsems[j]
                )

                @pl.when(blk + 64 < NBLK_FULL)
                def _():
                    pltpu.async_copy(
                        tT.at[:, pl.ds((blk + 64) * 128, 128)], slabs[j], rsems[j]
                    )

        return c

    lax.fori_loop(0, 123, step, 0)

    for j in range(2):
        pltpu.make_async_copy(slabPs[j], tab2.at[pl.ds(0, 64)], wsems[j]).wait()

    @pl.when(wid == NW - 1)
    def _():
        pltpu.sync_copy(tailp, tailv)
        pltpu.sync_copy(tailv, tab2.at[pl.ds(NBLK_FULL * 64, 40)])


def _pairify(tT, tailp):
    run = pl.kernel(
        _k1_body,
        out_type=jax.ShapeDtypeStruct((PAIR_PAD, 128), jnp.float32),
        mesh=_mesh,
        scratch_types=[pltpu.VMEM((EMBED, 128), jnp.float32) for _ in range(4)]
        + [pltpu.VMEM((40, 128), jnp.float32)]
        + [pltpu.SemaphoreType.DMA] * 4,
        compiler_params=pltpu.CompilerParams(needs_layout_passes=False),
    )
    return run(tT, tailp)


def _embed_body(wT, tab2, out, idxv, idx2v, pvv, bA, bB, tA, tB, gA, gB, wvA, wvB):
    bufs = (bA, bB)
    bufTs = (tA, tB)
    gsems = (gA, gB)
    wsems = (wvA, wvB)
    wid = lax.axis_index("s") * NC + lax.axis_index("c")
    b0 = wid * BBLK
    iota16, skews = _iota_skews()

    pltpu.sync_copy(wT.at[:, pl.ds(b0, BBLK)], idxv)

    @plsc.parallel_loop(0, SEQ, unroll=4)
    def prep(r):
        for k in range(BBLK // 16):
            v = idxv[r, pl.ds(16 * k, 16)]
            idx2v[r, pl.ds(16 * k, 16)] = lax.shift_right_logical(v, 1)
            pvv[r, pl.ds(16 * k, 16)] = lax.shift_left(lax.bitwise_and(v, 1), 6)

    def transpose_group(s, j):
        @plsc.parallel_loop(0, (EMBED // 16) * (BBLK // 16), unroll=4)
        def subtile(t):
            d0 = lax.mul(lax.bitwise_and(t, EMBED // 16 - 1), 16)
            jj0 = lax.mul(lax.shift_right_logical(t, 2), 16)
            ld_rows = iota16 + jj0
            pv = pvv[s, pl.ds(jj0, 16)]
            for c in range(16):
                sk_d = skews[c] + d0
                val = plsc.load_gather(bufs[j], [ld_rows, sk_d + pv])
                plsc.store_scatter(bufTs[j], [sk_d, ld_rows], val)

    for j in range(NB):
        pltpu.async_copy(tab2.at[idx2v.at[j]], bufs[j], gsems[j])

    def step(t, carry):
        for j in range(NB):
            s = t * NB + j
            pltpu.make_async_copy(tab2.at[idx2v.at[s]], bufs[j], gsems[j]).wait()

            @pl.when(s >= NB)
            def _():
                pltpu.make_async_copy(
                    bufTs[j], out.at[s - NB, :, pl.ds(b0, BBLK)], wsems[j]
                ).wait()

            transpose_group(s, j)
            pltpu.async_copy(bufTs[j], out.at[s, :, pl.ds(b0, BBLK)], wsems[j])

            @pl.when(s + NB < SEQ)
            def _():
                pltpu.async_copy(tab2.at[idx2v.at[s + NB]], bufs[j], gsems[j])
        return carry

    lax.fori_loop(0, SEQ // NB, step, 0)

    for j in range(NB):
        s_last = SEQ - NB + j
        pltpu.make_async_copy(
            bufTs[j], out.at[s_last, :, pl.ds(b0, BBLK)], wsems[j]
        ).wait()


@jax.jit
def _embed(wT, tab2):
    run = pl.kernel(
        _embed_body,
        out_type=jax.ShapeDtypeStruct((SEQ, EMBED, BATCH), jnp.float32),
        mesh=_mesh,
        scratch_types=[
            pltpu.VMEM((SEQ, BBLK), jnp.int32),
            pltpu.VMEM((SEQ, BBLK), jnp.int32),
            pltpu.VMEM((SEQ, BBLK), jnp.int32),
        ]
        + [pltpu.VMEM((BBLK, 128), jnp.float32) for _ in range(NB)]
        + [pltpu.VMEM((EMBED, BBLK), jnp.float32) for _ in range(NB)]
        + [pltpu.SemaphoreType.DMA] * (2 * NB),
        compiler_params=pltpu.CompilerParams(needs_layout_passes=False),
    )
    return run(wT, tab2)


def kernel(word_batch, table):
    wT = word_batch.astype(jnp.int32).T
    tableT = table.T
    tail = table[TAIL_V : VOCAB2 - 2].reshape(32, 128)
    tailp = jnp.concatenate([tail, jnp.zeros((8, 128), jnp.float32)], axis=0)
    tab2 = _pairify(tableT, tailp)
    outT = _embed(wT, tab2)
    return jnp.transpose(outT, (2, 0, 1))


# speedup vs baseline: 3.1420x; 1.4210x over previous
import jax
import jax.numpy as jnp
from jax import lax
from jax.experimental import pallas as pl
from jax.experimental.pallas import tpu as pltpu
from jax.experimental.pallas import tpu_sc as plsc

BATCH = 4096
SEQ = 200
EMBED = 64
VOCAB2 = 1000002

NC = 2
NS = 16
NW = NC * NS
BBLK = BATCH // NW
NB = 2

NBLK_FULL = (VOCAB2 // 2) // 64
TAIL_V = NBLK_FULL * 128
PAIR_PAD = NBLK_FULL * 64 + 40

_mesh = plsc.VectorSubcoreMesh(core_axis_name="c", subcore_axis_name="s")


def _iota_skews():
    iota16 = lax.iota(jnp.int32, 16)
    return iota16, [lax.bitwise_and(iota16 + c, 15) for c in range(16)]


K1R = 3


def _k1_body(tT, tailp, tab2, sA, sB, sC, pA, pB, pC, tailv,
             rA, rB, rC, wvA, wvB, wvC):
    slabs = (sA, sB, sC)
    slabPs = (pA, pB, pC)
    rsems = (rA, rB, rC)
    wsems = (wvA, wvB, wvC)
    wid = lax.axis_index("s") * NC + lax.axis_index("c")
    iota16, skews = _iota_skews()
    pcols = [
        lax.shift_left(lax.bitwise_and(skews[c], 1), 6) + iota16
        for c in range(16)
    ]

    def transpose_blk(j):
        @plsc.parallel_loop(0, 32, unroll=8)
        def subtile(t):
            r0 = lax.mul(lax.bitwise_and(t, 3), 16)
            c0 = lax.mul(lax.shift_right_logical(t, 2), 16)
            ld_rows = iota16 + r0
            for c in range(16):
                vcols = skews[c] + c0
                val = plsc.load_gather(slabs[j], [ld_rows, vcols])
                st_rows = lax.shift_right_logical(vcols, 1)
                st_cols = pcols[c] + r0
                plsc.store_scatter(slabPs[j], [st_rows, st_cols], val)

    for j in range(K1R):
        pltpu.async_copy(
            tT.at[:, pl.ds((32 * j + wid) * 128, 128)], slabs[j], rsems[j]
        )

    def step(tt, c):
        for j in range(K1R):
            t = K1R * tt + j
            blk = 32 * t + wid

            @pl.when(blk < NBLK_FULL)
            def _():
                pltpu.make_async_copy(
                    tT.at[:, pl.ds(blk * 128, 128)], slabs[j], rsems[j]
                ).wait()

                @pl.when(t >= K1R)
                def _():
                    pltpu.make_async_copy(
                        slabPs[j],
                        tab2.at[pl.ds((blk - 32 * K1R) * 64, 64)],
                        wsems[j],
                    ).wait()

                transpose_blk(j)
                pltpu.async_copy(
                    slabPs[j], tab2.at[pl.ds(blk * 64, 64)], wsems[j]
                )

                @pl.when(blk + 32 * K1R < NBLK_FULL)
                def _():
                    pltpu.async_copy(
                        tT.at[:, pl.ds((blk + 32 * K1R) * 128, 128)],
                        slabs[j],
                        rsems[j],
                    )

        return c

    lax.fori_loop(0, 246 // K1R + 1, step, 0)

    for j in range(K1R):
        pltpu.make_async_copy(slabPs[j], tab2.at[pl.ds(0, 64)], wsems[j]).wait()

    @pl.when(wid == NW - 1)
    def _():
        pltpu.sync_copy(tailp, tailv)
        pltpu.sync_copy(tailv, tab2.at[pl.ds(NBLK_FULL * 64, 40)])


def _pairify(tT, tailp):
    run = pl.kernel(
        _k1_body,
        out_type=jax.ShapeDtypeStruct((PAIR_PAD, 128), jnp.float32),
        mesh=_mesh,
        scratch_types=[pltpu.VMEM((EMBED, 128), jnp.float32) for _ in range(2 * K1R)]
        + [pltpu.VMEM((40, 128), jnp.float32)]
        + [pltpu.SemaphoreType.DMA] * (2 * K1R),
        compiler_params=pltpu.CompilerParams(needs_layout_passes=False),
    )
    return run(tT, tailp)


def _embed_body(wT, tab2, out, idxv, idx2v, pvv, bA, bB, tA, tB, gA, gB, wvA, wvB):
    bufs = (bA, bB)
    bufTs = (tA, tB)
    gsems = (gA, gB)
    wsems = (wvA, wvB)
    wid = lax.axis_index("s") * NC + lax.axis_index("c")
    b0 = wid * BBLK
    iota16, skews = _iota_skews()

    pltpu.sync_copy(wT.at[:, pl.ds(b0, BBLK)], idxv)

    @plsc.parallel_loop(0, SEQ, unroll=4)
    def prep(r):
        for k in range(BBLK // 16):
            v = idxv[r, pl.ds(16 * k, 16)]
            idx2v[r, pl.ds(16 * k, 16)] = lax.shift_right_logical(v, 1)
            pvv[r, pl.ds(16 * k, 16)] = lax.shift_left(lax.bitwise_and(v, 1), 6)

    def transpose_group(s, j):
        @plsc.parallel_loop(0, (EMBED // 16) * (BBLK // 16), unroll=8)
        def subtile(t):
            d0 = lax.mul(lax.bitwise_and(t, EMBED // 16 - 1), 16)
            jj0 = lax.mul(lax.shift_right_logical(t, 2), 16)
            ld_rows = iota16 + jj0
            pv = pvv[s, pl.ds(jj0, 16)]
            for c in range(16):
                sk_d = skews[c] + d0
                val = plsc.load_gather(bufs[j], [ld_rows, sk_d + pv])
                plsc.store_scatter(bufTs[j], [sk_d, ld_rows], val)

    for j in range(NB):
        pltpu.async_copy(tab2.at[idx2v.at[j]], bufs[j], gsems[j])

    def step(t, carry):
        for j in range(NB):
            s = t * NB + j
            pltpu.make_async_copy(tab2.at[idx2v.at[s]], bufs[j], gsems[j]).wait()

            @pl.when(s >= NB)
            def _():
                pltpu.make_async_copy(
                    bufTs[j], out.at[s - NB, :, pl.ds(b0, BBLK)], wsems[j]
                ).wait()

            transpose_group(s, j)
            pltpu.async_copy(bufTs[j], out.at[s, :, pl.ds(b0, BBLK)], wsems[j])

            @pl.when(s + NB < SEQ)
            def _():
                pltpu.async_copy(tab2.at[idx2v.at[s + NB]], bufs[j], gsems[j])
        return carry

    lax.fori_loop(0, SEQ // NB, step, 0)

    for j in range(NB):
        s_last = SEQ - NB + j
        pltpu.make_async_copy(
            bufTs[j], out.at[s_last, :, pl.ds(b0, BBLK)], wsems[j]
        ).wait()


@jax.jit
def _embed(wT, tab2):
    run = pl.kernel(
        _embed_body,
        out_type=jax.ShapeDtypeStruct((SEQ, EMBED, BATCH), jnp.float32),
        mesh=_mesh,
        scratch_types=[
            pltpu.VMEM((SEQ, BBLK), jnp.int32),
            pltpu.VMEM((SEQ, BBLK), jnp.int32),
            pltpu.VMEM((SEQ, BBLK), jnp.int32),
        ]
        + [pltpu.VMEM((BBLK, 128), jnp.float32) for _ in range(NB)]
        + [pltpu.VMEM((EMBED, BBLK), jnp.float32) for _ in range(NB)]
        + [pltpu.SemaphoreType.DMA] * (2 * NB),
        compiler_params=pltpu.CompilerParams(needs_layout_passes=False),
    )
    return run(wT, tab2)


def kernel(word_batch, table):
    wT = word_batch.astype(jnp.int32).T
    tableT = table.T
    tail = table[TAIL_V : VOCAB2 - 2].reshape(32, 128)
    tailp = jnp.concatenate([tail, jnp.zeros((8, 128), jnp.float32)], axis=0)
    tab2 = _pairify(tableT, tailp)
    outT = _embed(wT, tab2)
    return jnp.transpose(outT, (2, 0, 1))
